# baseline (speedup 1.0000x reference)
import jax
import jax.numpy as jnp
from jax.experimental import pallas as pl
from jax.experimental.pallas import tpu as pltpu

_BM = 400


def _support_body(x_ref, w0_ref, out_ref):
    out_ref[...] = jnp.dot(x_ref[...], w0_ref[...],
                           preferred_element_type=jnp.float32)


def _layer1_step(adj, sup_ref, b0_ref, w1_ref, s2_ref, row0, bm):
    acc = jnp.dot(adj, sup_ref[...], preferred_element_type=jnp.float32)
    h = acc + sup_ref[pl.ds(row0, bm), :] + b0_ref[...]
    h = h - jnp.mean(h, axis=1, keepdims=True)
    nrm = jnp.sqrt(jnp.sum(h * h, axis=1, keepdims=True))
    h = h / (nrm + 1e-6)
    h = jnp.maximum(h, 0.0)
    s2_ref[pl.ds(row0, bm), :] = jnp.dot(
        h, w1_ref[...], preferred_element_type=jnp.float32)


def _layer2_step(adj, s2_ref, b1_ref, out_ref, row0, bm):
    logits = jnp.dot(adj, s2_ref[...], preferred_element_type=jnp.float32)
    logits = logits + s2_ref[pl.ds(row0, bm), :] + b1_ref[...]
    m = jnp.max(logits, axis=1, keepdims=True)
    sh = logits - m
    lse = jnp.log(jnp.sum(jnp.exp(sh), axis=1, keepdims=True))
    out_ref[pl.ds(row0, bm), :] = sh - lse


def kernel(x, adj, W0, b0, W1, b1):
    n, nfeat = x.shape
    nhid = W0.shape[1]
    nclass = W1.shape[1]
    bm = _BM if (n % _BM == 0 and n % 8 == 0) else n
    nb = n // bm
    b0r = b0.reshape(1, nhid)
    b1r = b1.reshape(1, nclass)

    def adj_idx(s):
        return (jnp.where(s < nb, s, s - nb), 0)

    support = pl.pallas_call(
        _support_body,
        out_shape=jax.ShapeDtypeStruct((n, nhid), jnp.float32),
    )(x, W0)

    def outer_body(adj_hbm, sup_ref, b0_ref, w1_ref, b1_ref,
                   out_ref, s2_ref):
        def inner(adj_ref):
            s = pl.program_id(0)

            @pl.when(s < nb)
            def _():
                _layer1_step(adj_ref[...], sup_ref, b0_ref, w1_ref,
                             s2_ref, s * bm, bm)

            @pl.when(s >= nb)
            def _():
                _layer2_step(adj_ref[...], s2_ref, b1_ref, out_ref,
                             (s - nb) * bm, bm)

        pltpu.emit_pipeline(
            inner,
            grid=(2 * nb,),
            in_specs=[pl.BlockSpec((bm, n), adj_idx,
                                   pipeline_mode=pl.Buffered(buffer_count=3))],
        )(adj_hbm)

    return pl.pallas_call(
        outer_body,
        in_specs=[
            pl.BlockSpec(memory_space=pl.ANY),
            pl.BlockSpec((n, nhid), lambda: (0, 0)),
            pl.BlockSpec((1, nhid), lambda: (0, 0)),
            pl.BlockSpec((nhid, nclass), lambda: (0, 0)),
            pl.BlockSpec((1, nclass), lambda: (0, 0)),
        ],
        out_specs=pl.BlockSpec((n, nclass), lambda: (0, 0)),
        out_shape=jax.ShapeDtypeStruct((n, nclass), jnp.float32),
        scratch_shapes=[
            pltpu.VMEM((n, nclass), jnp.float32),
        ],
        compiler_params=pltpu.CompilerParams(
            vmem_limit_bytes=64 * 1024 * 1024),
    )(adj, support, b0r, W1, b1r)


# speedup vs baseline: 1.0566x; 1.0566x over previous
import jax
import jax.numpy as jnp
from jax.experimental import pallas as pl
from jax.experimental.pallas import tpu as pltpu

_BM = 400


def _fused_body(adj_ref, x_ref, w0_ref, b0_ref, w1_ref, b1_ref,
                out_ref, sup_ref, s2_ref):
    s = pl.program_id(0)
    nb = pl.num_programs(0) // 2
    bm = adj_ref.shape[0]

    @pl.when(s == 0)
    def _():
        sup_ref[...] = jnp.dot(x_ref[...], w0_ref[...],
                               preferred_element_type=jnp.float32)

    @pl.when(s < nb)
    def _():
        row0 = s * bm
        acc = jnp.dot(adj_ref[...], sup_ref[...],
                      preferred_element_type=jnp.float32)
        h = acc + sup_ref[pl.ds(row0, bm), :] + b0_ref[...]
        h = h - jnp.mean(h, axis=1, keepdims=True)
        nrm = jnp.sqrt(jnp.sum(h * h, axis=1, keepdims=True))
        h = h / (nrm + 1e-6)
        h = jnp.maximum(h, 0.0)
        s2_ref[pl.ds(row0, bm), :] = jnp.dot(
            h, w1_ref[...], preferred_element_type=jnp.float32)

    @pl.when(s >= nb)
    def _():
        row0 = (s - nb) * bm
        logits = jnp.dot(adj_ref[...], s2_ref[...],
                         preferred_element_type=jnp.float32)
        logits = logits + s2_ref[pl.ds(row0, bm), :] + b1_ref[...]
        m = jnp.max(logits, axis=1, keepdims=True)
        sh = logits - m
        lse = jnp.log(jnp.sum(jnp.exp(sh), axis=1, keepdims=True))
        out_ref[pl.ds(row0, bm), :] = sh - lse


def kernel(x, adj, W0, b0, W1, b1):
    n, nfeat = x.shape
    nhid = W0.shape[1]
    nclass = W1.shape[1]
    bm = _BM if (n % _BM == 0 and n % 8 == 0) else n
    nb = n // bm
    b0r = b0.reshape(1, nhid)
    b1r = b1.reshape(1, nclass)

    def adj_idx(s):
        return (jnp.where(s < nb, s, s - nb), 0)

    return pl.pallas_call(
        _fused_body,
        grid=(2 * nb,),
        in_specs=[
            pl.BlockSpec((bm, n), adj_idx),
            pl.BlockSpec((n, nfeat), lambda s: (0, 0)),
            pl.BlockSpec((nfeat, nhid), lambda s: (0, 0)),
            pl.BlockSpec((1, nhid), lambda s: (0, 0)),
            pl.BlockSpec((nhid, nclass), lambda s: (0, 0)),
            pl.BlockSpec((1, nclass), lambda s: (0, 0)),
        ],
        out_specs=pl.BlockSpec((n, nclass), lambda s: (0, 0)),
        out_shape=jax.ShapeDtypeStruct((n, nclass), jnp.float32),
        scratch_shapes=[
            pltpu.VMEM((n, nhid), jnp.float32),
            pltpu.VMEM((n, nclass), jnp.float32),
        ],
        compiler_params=pltpu.CompilerParams(
            dimension_semantics=("arbitrary",)),
    )(adj, x, W0, b0r, W1, b1r)
